# speedup vs baseline: 1.5396x; 1.5396x over previous
import jax
import jax.numpy as jnp
from jax.experimental import pallas as pl
from jax.experimental.pallas import tpu as pltpu


def _proj_body(x_ref, w_ref, as_ref, ad_ref, h_ref, asum_ref, adsum_ref):
    h = jnp.dot(x_ref[...], w_ref[...], preferred_element_type=jnp.float32)
    h_ref[...] = h
    asum_ref[...] = jnp.sum(h * as_ref[...], axis=-1, keepdims=True)
    adsum_ref[...] = jnp.sum(h * ad_ref[...], axis=-1, keepdims=True)


def _project(x, W, att_src, att_dst):
    n, d_in = x.shape
    d_out = W.shape[1]
    blk = 1000
    grid = (n // blk,)
    h, a_s, a_d = pl.pallas_call(
        _proj_body,
        grid=grid,
        in_specs=[
            pl.BlockSpec((blk, d_in), lambda i: (i, 0)),
            pl.BlockSpec((d_in, d_out), lambda i: (0, 0)),
            pl.BlockSpec((1, d_out), lambda i: (0, 0)),
            pl.BlockSpec((1, d_out), lambda i: (0, 0)),
        ],
        out_specs=[
            pl.BlockSpec((blk, d_out), lambda i: (i, 0)),
            pl.BlockSpec((blk, 1), lambda i: (i, 0)),
            pl.BlockSpec((blk, 1), lambda i: (i, 0)),
        ],
        out_shape=[
            jax.ShapeDtypeStruct((n, d_out), jnp.float32),
            jax.ShapeDtypeStruct((n, 1), jnp.float32),
            jax.ShapeDtypeStruct((n, 1), jnp.float32),
        ],
    )(x, W, att_src.reshape(1, -1), att_dst.reshape(1, -1))
    return h, a_s[:, 0], a_d[:, 0]


def kernel(x, edge_index, W, att_src, att_dst, bias):
    n = x.shape[0]
    src = edge_index[0]
    dst = edge_index[1]
    loop = jnp.arange(n, dtype=edge_index.dtype)
    src = jnp.concatenate([src, loop])
    dst = jnp.concatenate([dst, loop])
    h, a_s, a_d = _project(x, W, att_src, att_dst)
    e = a_s[src] + a_d[dst]
    e = jax.nn.leaky_relu(e, 0.2)
    ex = jnp.exp(e)
    denom = jax.ops.segment_sum(ex, dst, num_segments=n)
    alpha = ex / (denom[dst] + 1e-16)
    msg = h[src] * alpha[:, None]
    out = jax.ops.segment_sum(msg, dst, num_segments=n)
    out = out + bias
    return jax.nn.relu(out)


# speedup vs baseline: 16.3085x; 10.5930x over previous
import functools

import jax
import jax.numpy as jnp
from jax import lax
from jax.experimental import pallas as pl
from jax.experimental.pallas import tpu as pltpu
from jax.experimental.pallas import tpu_sc as plsc

N = 10000
NP = 10240
D = 128
NR = NP // D
E_TOTAL = 320000 + N
EW = 10368
WORKERS = 32
EP = WORKERS * EW
NSUB = 16
ROWS_PER_SUB = NP // NSUB


def _proj_body(x_ref, w_ref, as_ref, ad_ref, h_ref, asum_ref, adsum_ref):
    h = jnp.dot(x_ref[...], w_ref[...], preferred_element_type=jnp.float32)
    h_ref[...] = h
    asum_ref[...] = jnp.sum(h * as_ref[...], axis=-1, keepdims=True)
    adsum_ref[...] = jnp.sum(h * ad_ref[...], axis=-1, keepdims=True)


def _project(x_pad, W, att_src, att_dst):
    blk = 1024
    grid = (NP // blk,)
    h, a_s, a_d = pl.pallas_call(
        _proj_body,
        grid=grid,
        in_specs=[
            pl.BlockSpec((blk, D), lambda i: (i, 0)),
            pl.BlockSpec((D, D), lambda i: (0, 0)),
            pl.BlockSpec((1, D), lambda i: (0, 0)),
            pl.BlockSpec((1, D), lambda i: (0, 0)),
        ],
        out_specs=[
            pl.BlockSpec((blk, D), lambda i: (i, 0)),
            pl.BlockSpec((blk, 1), lambda i: (i, 0)),
            pl.BlockSpec((blk, 1), lambda i: (i, 0)),
        ],
        out_shape=[
            jax.ShapeDtypeStruct((NP, D), jnp.float32),
            jax.ShapeDtypeStruct((NP, 1), jnp.float32),
            jax.ShapeDtypeStruct((NP, 1), jnp.float32),
        ],
    )(x_pad, W, att_src.reshape(1, -1), att_dst.reshape(1, -1))
    return h, a_s.reshape(NP), a_d.reshape(NP)


def _zero_2d(ref, rows):
    z = jnp.zeros((16,), jnp.float32)

    def body(i, _):
        r = i // 8
        c = (i % 8) * 16
        ref[r, pl.ds(c, 16)] = z
        return 0

    lax.fori_loop(0, rows * 8, body, 0)


def _sc1_body(as_hbm, ad_hbm, src_hbm, dst_hbm, ex_hbm, den_hbm,
              asv, adv, srcv, dstv, exv, denm, idrows, outbuf, den_sh, sem):
    cid = lax.axis_index("c")
    sid = lax.axis_index("s")
    wid = cid * NSUB + sid
    base = wid * EW

    pltpu.sync_copy(as_hbm, asv)
    pltpu.sync_copy(ad_hbm, adv)
    pltpu.sync_copy(src_hbm.at[pl.ds(base, EW)], srcv)
    pltpu.sync_copy(dst_hbm.at[pl.ds(base, EW)], dstv)

    _zero_2d(denm, NR)

    @pl.when(sid == 0)
    def _():
        pltpu.sync_copy(denm, den_sh)

    def idbody(i, _):
        idrows[0, pl.ds(i * 16, 16)] = (
            lax.iota(jnp.int32, 16) + i * 16)
        return 0
    lax.fori_loop(0, NR // 16, idbody, 0)

    plsc.subcore_barrier()

    def body(i, _):
        off = i * 16
        s_idx = srcv[pl.ds(off, 16)]
        d_idx = dstv[pl.ds(off, 16)]
        s_hi = lax.shift_right_logical(s_idx, 7)
        s_lo = lax.bitwise_and(s_idx, 127)
        d_hi = lax.shift_right_logical(d_idx, 7)
        d_lo = lax.bitwise_and(d_idx, 127)
        e = (plsc.load_gather(asv, [s_hi, s_lo])
             + plsc.load_gather(adv, [d_hi, d_lo]))
        e = jnp.where(e >= 0.0, e, 0.2 * e)
        ex = jnp.exp(e)
        exv[pl.ds(off, 16)] = ex
        plsc.addupdate_scatter(denm, [d_hi, d_lo], ex)
        return 0

    lax.fori_loop(0, EW // 16, body, 0)

    pltpu.sync_copy(exv, ex_hbm.at[pl.ds(base, EW)])
    pltpu.sync_copy(denm, den_sh.at[idrows.at[0]], add=True)
    plsc.subcore_barrier()
    @pl.when(sid < NR // 8)
    def _():
        pltpu.sync_copy(den_sh.at[pl.ds(sid * 8, 8)], outbuf)
        pltpu.sync_copy(outbuf, den_hbm.at[cid, pl.ds(sid * 8, 8)])


def _sc1(a_s, a_d, srcp, dstp):
    mesh = plsc.VectorSubcoreMesh(core_axis_name="c", subcore_axis_name="s")
    f = pl.kernel(
        _sc1_body,
        out_type=[
            jax.ShapeDtypeStruct((EP,), jnp.float32),
            jax.ShapeDtypeStruct((2, NR, D), jnp.float32),
        ],
        mesh=mesh,
        compiler_params=pltpu.CompilerParams(needs_layout_passes=False),
        scratch_types=[
            pltpu.VMEM((NR, D), jnp.float32),
            pltpu.VMEM((NR, D), jnp.float32),
            pltpu.VMEM((EW,), jnp.int32),
            pltpu.VMEM((EW,), jnp.int32),
            pltpu.VMEM((EW,), jnp.float32),
            pltpu.VMEM((NR, D), jnp.float32),
            pltpu.VMEM((1, NR), jnp.int32),
            pltpu.VMEM((8, D), jnp.float32),
            pltpu.VMEM_SHARED((NR, D), jnp.float32),
            pltpu.SemaphoreType.DMA,
        ],
    )
    return f(a_s.reshape(NR, D), a_d.reshape(NR, D), srcp, dstp)


_GDN = lax.GatherDimensionNumbers(
    offset_dims=(), collapsed_slice_dims=(0,), start_index_map=(0,))


def _bcast_lane(vec, k):
    idx = jnp.full((16, 1), k, jnp.int32)
    return lax.gather(vec, idx, _GDN, slice_sizes=(1,),
                      mode=lax.GatherScatterMode.PROMISE_IN_BOUNDS)


G = 16
CH = 2592
NG = CH // G


def _sc2_body(h_hbm, ex_hbm, invd_hbm, src_hbm, dst_hbm, out_hbm,
              srcv, dstv, exv, invd, rows, ala, alb, out_sh, sem):
    cid = lax.axis_index("c")
    sid = lax.axis_index("s")
    wid = cid * NSUB + sid
    base = wid * EW

    _zero_2d(invd, NR)
    for t in range(ROWS_PER_SUB // NR):
        pltpu.sync_copy(invd, out_sh.at[pl.ds(sid * ROWS_PER_SUB + t * NR, NR)])
    plsc.subcore_barrier()
    pltpu.sync_copy(invd_hbm, invd)

    def chunk_body(ci, _):
        cbase = base + ci * CH
        pltpu.sync_copy(src_hbm.at[pl.ds(cbase, CH)], srcv)
        pltpu.sync_copy(dst_hbm.at[pl.ds(cbase, CH)], dstv)
        pltpu.sync_copy(ex_hbm.at[pl.ds(cbase, CH)], exv)

        def compute_alpha(g):
            off = g * G
            d_idx = dstv[pl.ds(off, G)]
            ex = exv[pl.ds(off, G)]
            d_hi = lax.shift_right_logical(d_idx, 7)
            d_lo = lax.bitwise_and(d_idx, 127)
            return ex * plsc.load_gather(invd, [d_hi, d_lo])

        ala[0, :] = compute_alpha(0)

        def body2(i2, _):
            for rd, wr, half in ((ala, alb, 0), (alb, ala, 1)):
                g = i2 * 2 + half
                off = g * G
                s_idx = srcv[pl.ds(off, G)]
                d_idx = dstv[pl.ds(off, G)]
                zz = d_idx * 0
                pltpu.async_copy(h_hbm.at[s_idx], rows, sem).wait()
                wr[0, :] = compute_alpha(jnp.minimum(g + 1, NG - 1))
                alpha = rd[0, :]
                for k in range(G):
                    ab = lax.gather(alpha, (zz + k).reshape(G, 1), _GDN,
                                    slice_sizes=(1,),
                                    mode=lax.GatherScatterMode.PROMISE_IN_BOUNDS)
                    for j in range(8):
                        c = j * 16
                        rows[k, pl.ds(c, 16)] = rows[k, pl.ds(c, 16)] * ab
                pltpu.sync_copy(rows, out_sh.at[d_idx], add=True)
            return 0

        lax.fori_loop(0, NG // 2, body2, 0)
        return 0

    lax.fori_loop(0, EW // CH, chunk_body, 0)
    plsc.subcore_barrier()
    pltpu.sync_copy(out_sh.at[pl.ds(sid * ROWS_PER_SUB, ROWS_PER_SUB)],
                    out_hbm.at[cid, pl.ds(sid * ROWS_PER_SUB, ROWS_PER_SUB)])


def _sc2(h, ex_all, invd, srcp, dstp):
    mesh = plsc.VectorSubcoreMesh(core_axis_name="c", subcore_axis_name="s")
    f = pl.kernel(
        _sc2_body,
        out_type=[
            jax.ShapeDtypeStruct((2, NP, D), jnp.float32),
        ],
        mesh=mesh,
        compiler_params=pltpu.CompilerParams(needs_layout_passes=False),
        scratch_types=[
            pltpu.VMEM((CH,), jnp.int32),
            pltpu.VMEM((CH,), jnp.int32),
            pltpu.VMEM((CH,), jnp.float32),
            pltpu.VMEM((NR, D), jnp.float32),
            pltpu.VMEM((G, D), jnp.float32),
            pltpu.VMEM((1, G), jnp.float32),
            pltpu.VMEM((1, G), jnp.float32),
            pltpu.VMEM_SHARED((NP, D), jnp.float32),
            pltpu.SemaphoreType.DMA,
        ],
    )
    return f(h, ex_all, invd, srcp, dstp)


def _invd_body(den_ref, invd_ref):
    invd_ref[...] = 1.0 / (den_ref[0] + den_ref[1] + 1e-16)


def _invd(den2):
    return pl.pallas_call(
        _invd_body,
        out_shape=jax.ShapeDtypeStruct((NR, D), jnp.float32),
    )(den2)


def _combine_body(p0_ref, p1_ref, b_ref, o_ref):
    o_ref[...] = jax.nn.relu(p0_ref[...] + p1_ref[...] + b_ref[...])


def _combine(p0, p1, bias):
    blk = 1000
    return pl.pallas_call(
        _combine_body,
        grid=(N // blk,),
        in_specs=[
            pl.BlockSpec((blk, D), lambda i: (i, 0)),
            pl.BlockSpec((blk, D), lambda i: (i, 0)),
            pl.BlockSpec((1, D), lambda i: (0, 0)),
        ],
        out_specs=pl.BlockSpec((blk, D), lambda i: (i, 0)),
        out_shape=jax.ShapeDtypeStruct((N, D), jnp.float32),
    )(p0, p1, bias.reshape(1, D))


def kernel(x, edge_index, W, att_src, att_dst, bias):
    src = edge_index[0]
    dst = edge_index[1]
    loop = jnp.arange(N, dtype=jnp.int32)
    pad = EP - E_TOTAL
    srcp = jnp.concatenate([src, loop, jnp.zeros((pad,), jnp.int32)])
    dstp = jnp.concatenate([dst, loop, jnp.full((pad,), NP - 1, jnp.int32)])

    x_pad = jnp.pad(x, ((0, NP - N), (0, 0)))
    h, a_s, a_d = _project(x_pad, W, att_src, att_dst)

    ex_all, den2 = _sc1(a_s, a_d, srcp, dstp)
    invd = _invd(den2)
    (out2,) = _sc2(h, ex_all, invd, srcp, dstp)
    return _combine(out2[0, :N], out2[1, :N], bias)


def _sc2g_body(h_hbm, src_hbm, dst_hbm, ex_hbm, invd_hbm, msg_hbm, al_hbm,
               srcv, dstv, exv, invd, rows, alphav, sem):
    cid = lax.axis_index("c")
    sid = lax.axis_index("s")
    wid = cid * NSUB + sid
    base = wid * EW
    pltpu.sync_copy(src_hbm.at[pl.ds(base, EW)], srcv)
    pltpu.sync_copy(dst_hbm.at[pl.ds(base, EW)], dstv)
    pltpu.sync_copy(ex_hbm.at[pl.ds(base, EW)], exv)
    pltpu.sync_copy(invd_hbm, invd)

    def body(i, _):
        off = i * G
        s_idx = srcv[pl.ds(off, G)]
        d_idx = dstv[pl.ds(off, G)]
        ex = exv[pl.ds(off, G)]
        d_hi = lax.shift_right_logical(d_idx, 7)
        d_lo = lax.bitwise_and(d_idx, 127)
        alpha = ex * plsc.load_gather(invd, [d_hi, d_lo])
        alphav[pl.ds(off, G)] = alpha
        pltpu.async_copy(h_hbm.at[s_idx], rows, sem).wait()
        pltpu.sync_copy(rows, msg_hbm.at[pl.ds(base + off, G)])
        return 0

    lax.fori_loop(0, EW // G, body, 0)
    pltpu.sync_copy(alphav, al_hbm.at[pl.ds(base, EW)])


def _sc2g(h, srcp, dstp, ex_all, invd):
    mesh = plsc.VectorSubcoreMesh(core_axis_name="c", subcore_axis_name="s")
    f = pl.kernel(
        _sc2g_body,
        out_type=[jax.ShapeDtypeStruct((EP, D), jnp.float32),
                  jax.ShapeDtypeStruct((EP,), jnp.float32)],
        mesh=mesh,
        compiler_params=pltpu.CompilerParams(needs_layout_passes=False),
        scratch_types=[
            pltpu.VMEM((EW,), jnp.int32),
            pltpu.VMEM((EW,), jnp.int32),
            pltpu.VMEM((EW,), jnp.float32),
            pltpu.VMEM((NR, D), jnp.float32),
            pltpu.VMEM((G, D), jnp.float32),
            pltpu.VMEM((EW,), jnp.float32),
            pltpu.SemaphoreType.DMA,
        ],
    )
    return f(h, srcp, dstp, ex_all, invd)
